# baseline (speedup 1.0000x reference)
import functools

import jax
import jax.numpy as jnp
from jax import lax
from jax.experimental import pallas as pl
from jax.experimental.pallas import tpu as pltpu
from jax.experimental.pallas import tpu_sc as plsc

_LANES = 16
_NWORKERS = 32
_CHUNK = 16384
_NBUF = 4
_UNROLL = 16


def _sc_partial_sums(atomic_numbers, table):
    natoms = atomic_numbers.shape[0]
    nrefs = table.shape[0]
    per_w = natoms // _NWORKERS
    nchunks = per_w // _CHUNK
    assert per_w % _CHUNK == 0

    mesh = plsc.VectorSubcoreMesh(core_axis_name="c", subcore_axis_name="s")

    @functools.partial(
        pl.kernel,
        mesh=mesh,
        out_type=jax.ShapeDtypeStruct((_NWORKERS, _LANES), jnp.float32),
        compiler_params=pltpu.CompilerParams(
            use_tc_tiling_on_sc=False, needs_layout_passes=False
        ),
        scratch_types=[
            [pltpu.VMEM((_CHUNK,), jnp.int32) for _ in range(_NBUF)],
            pltpu.VMEM((nrefs * _LANES,), jnp.float32),
            pltpu.VMEM((nrefs,), jnp.float32),
            pltpu.VMEM((_LANES,), jnp.float32),
            [pltpu.SemaphoreType.DMA for _ in range(_NBUF)],
        ],
    )
    def k(an_hbm, tab_hbm, out_hbm, bufs, hist, tab_v, acc_v, sems):
        wid = lax.axis_index("s") * 2 + lax.axis_index("c")
        base = wid * per_w

        pltpu.sync_copy(tab_hbm, tab_v)
        zeros = jnp.zeros((_LANES,), jnp.float32)
        for b in range(nrefs):
            hist[pl.ds(b * _LANES, _LANES)] = zeros

        lanes = lax.iota(jnp.int32, _LANES)
        ones = jnp.ones((_LANES,), jnp.float32)

        def start(c):
            return pltpu.async_copy(
                an_hbm.at[pl.ds(base + c * _CHUNK, _CHUNK)],
                bufs[c % _NBUF],
                sems[c % _NBUF],
            )

        def process(buf):
            def body(i, carry):
                start_i = i * (_UNROLL * _LANES)
                idxs = [
                    buf[pl.ds(start_i + u * _LANES, _LANES)]
                    for u in range(_UNROLL)
                ]
                addrs = [ix * _LANES + lanes for ix in idxs]
                for a in addrs:
                    plsc.addupdate_scatter(hist, [a], ones)
                return carry

            lax.fori_loop(0, _CHUNK // (_UNROLL * _LANES), body, 0)

        handles = [start(c) for c in range(_NBUF - 1)]
        for c in range(nchunks):
            if c + _NBUF - 1 < nchunks:
                handles.append(start(c + _NBUF - 1))
            handles[c].wait()
            process(bufs[c % _NBUF])

        acc = jnp.zeros((_LANES,), jnp.float32)
        for b in range(nrefs):
            tab_b = plsc.load_gather(tab_v, [jnp.full((_LANES,), b, jnp.int32)])
            acc = acc + hist[pl.ds(b * _LANES, _LANES)] * tab_b
        acc_v[...] = acc
        pltpu.sync_copy(acc_v, out_hbm.at[wid])

    return k(atomic_numbers, table)


def _tc_combine(partials_ref, te_ref, out_ref):
    out_ref[...] = te_ref[...] - jnp.sum(partials_ref[...])


def kernel(total_energy, atomic_numbers, per_atom_references):
    an = atomic_numbers.astype(jnp.int32)
    tab = per_atom_references.astype(jnp.float32)

    partials = _sc_partial_sums(an, tab)

    return pl.pallas_call(
        _tc_combine,
        out_shape=jax.ShapeDtypeStruct(total_energy.shape, jnp.float32),
    )(partials, total_energy)


# speedup vs baseline: 1.0241x; 1.0241x over previous
import functools

import jax
import jax.numpy as jnp
from jax import lax
from jax.experimental import pallas as pl
from jax.experimental.pallas import tpu as pltpu
from jax.experimental.pallas import tpu_sc as plsc

_LANES = 16
_NWORKERS = 32
_CHUNK = 32768
_NBUF = 2
_UNROLL = 16


def _sc_partial_sums(atomic_numbers, table_bcast):
    natoms = atomic_numbers.shape[0]
    nrefs = table_bcast.shape[0] // _LANES
    per_w = natoms // _NWORKERS
    nchunks = per_w // _CHUNK
    assert per_w % _CHUNK == 0

    mesh = plsc.VectorSubcoreMesh(core_axis_name="c", subcore_axis_name="s")

    @functools.partial(
        pl.kernel,
        mesh=mesh,
        out_type=jax.ShapeDtypeStruct((_NWORKERS, _LANES), jnp.float32),
        compiler_params=pltpu.CompilerParams(
            use_tc_tiling_on_sc=False, needs_layout_passes=False
        ),
        scratch_types=[
            [pltpu.VMEM((_CHUNK,), jnp.int32) for _ in range(_NBUF)],
            pltpu.VMEM((nrefs * _LANES,), jnp.float32),
            pltpu.VMEM((nrefs * _LANES,), jnp.float32),
            pltpu.VMEM((_LANES,), jnp.float32),
            [pltpu.SemaphoreType.DMA for _ in range(_NBUF)],
        ],
    )
    def k(an_hbm, tab_hbm, out_hbm, bufs, hist, tab_v, acc_v, sems):
        wid = lax.axis_index("s") * 2 + lax.axis_index("c")
        base = wid * per_w

        pltpu.sync_copy(tab_hbm, tab_v)
        zeros = jnp.zeros((_LANES,), jnp.float32)
        for b in range(nrefs):
            hist[pl.ds(b * _LANES, _LANES)] = zeros

        lanes = lax.iota(jnp.int32, _LANES)
        ones = jnp.ones((_LANES,), jnp.float32)

        def start(c):
            return pltpu.async_copy(
                an_hbm.at[pl.ds(base + c * _CHUNK, _CHUNK)],
                bufs[c % _NBUF],
                sems[c % _NBUF],
            )

        def process(buf):
            def body(i, carry):
                start_i = i * (_UNROLL * _LANES)
                idxs = [
                    buf[pl.ds(start_i + u * _LANES, _LANES)]
                    for u in range(_UNROLL)
                ]
                addrs = [ix * _LANES + lanes for ix in idxs]
                for a in addrs:
                    plsc.addupdate_scatter(hist, [a], ones)
                return carry

            lax.fori_loop(0, _CHUNK // (_UNROLL * _LANES), body, 0)

        handles = [start(c) for c in range(_NBUF - 1)]
        for c in range(nchunks):
            if c + _NBUF - 1 < nchunks:
                handles.append(start(c + _NBUF - 1))
            handles[c].wait()
            process(bufs[c % _NBUF])

        acc = jnp.zeros((_LANES,), jnp.float32)
        for b in range(nrefs):
            acc = acc + hist[pl.ds(b * _LANES, _LANES)] * tab_v[pl.ds(b * _LANES, _LANES)]
        acc_v[...] = acc
        pltpu.sync_copy(acc_v, out_hbm.at[wid])

    return k(atomic_numbers, table_bcast)


def _tc_combine(partials_ref, te_ref, out_ref):
    out_ref[...] = te_ref[...] - jnp.sum(partials_ref[...])


def kernel(total_energy, atomic_numbers, per_atom_references):
    an = atomic_numbers.astype(jnp.int32)
    nrefs = per_atom_references.shape[0]
    table_bcast = (
        jnp.broadcast_to(per_atom_references[:, None], (nrefs, _LANES))
        .astype(jnp.float32)
        .reshape(nrefs * _LANES)
    )

    partials = _sc_partial_sums(an, table_bcast)

    return pl.pallas_call(
        _tc_combine,
        out_shape=jax.ShapeDtypeStruct(total_energy.shape, jnp.float32),
    )(partials, total_energy)
